# baseline (speedup 1.0000x reference)
import functools

import jax
import jax.numpy as jnp
from jax import lax
from jax.experimental import pallas as pl
from jax.experimental.pallas import tpu as pltpu
from jax.experimental.pallas import tpu_sc as plsc

_NF = 64
_ND = 100000
_B = 1024
_BN = 2048
_NB = _ND // _BN
_TAIL = _ND - _NB * _BN
_NG = _BN // 128
_INV_TEMP = 20.0
_LN2 = 0.6931471805599453
_LOG2E = 1.4426950408889634
_SHIFT = 100.0

_NC = 2
_NS = 16
_L = 16
_NW = _NC * _NS
_BPW = _B // _NW


def _stream_kernel(x_ref, ft_ref, c_ref, blk_ref, grp_ref,
                   s_ref, p_ref, buf_ref):
    j = pl.program_id(0)

    @pl.when(j == 0)
    def _init():
        s_ref[...] = jnp.zeros((_B, 128), jnp.float32)
        p_ref[...] = jnp.zeros((_B, 128), jnp.float32)

    x = x_ref[...]

    prev = buf_ref[...]
    e = jnp.exp2(prev - c_ref[...])
    parts = [e[:, k * 128:(k + 1) * 128] for k in range(_NG)]
    while len(parts) > 1:
        parts = [a + b for a, b in zip(parts[::2], parts[1::2])]
    s_ref[...] += jnp.where(j > 0, parts[0], 0.0)

    gate = blk_ref[...] == (j - 1)
    grp = grp_ref[...]
    pacc = p_ref[...]
    for k in range(_NG):
        pacc = jnp.where(gate & (grp == k),
                         prev[:, k * 128:(k + 1) * 128], pacc)
    p_ref[...] = pacc

    ft = ft_ref[...]
    logits = lax.dot_general(
        x.astype(jnp.bfloat16), ft.astype(jnp.bfloat16),
        (((1,), (0,)), ((), ())),
        preferred_element_type=jnp.float32)
    buf_ref[...] = logits


def _finalize_kernel(x_ref, ft_ref, c_ref, tt_ref, blk_ref, lane_ref,
                     s_ref, p_ref, out_ref):
    x = x_ref[...]
    c = c_ref[...]

    ft = ft_ref[...]
    logits = lax.dot_general(
        x.astype(jnp.bfloat16), ft.astype(jnp.bfloat16),
        (((1,), (0,)), ((), ())),
        preferred_element_type=jnp.float32)
    col = lax.broadcasted_iota(jnp.int32, (_B, _BN), 1)
    e = jnp.where(col < _TAIL, jnp.exp2(logits - c), 0.0)
    lanes = s_ref[...] + sum(
        e[:, k * 128:(k + 1) * 128] for k in range(_NG))
    s = jnp.sum(lanes, axis=1, keepdims=True)

    tt = tt_ref[...]
    picked_tail = jnp.sum(jnp.where(col == tt, logits, 0.0),
                          axis=1, keepdims=True)
    lane128 = lax.broadcasted_iota(jnp.int32, (_B, 128), 1)
    picked_p = jnp.sum(jnp.where(lane128 == lane_ref[...], p_ref[...], 0.0),
                       axis=1, keepdims=True)
    picked = jnp.where(blk_ref[...] == _NB, picked_tail, picked_p)

    logz = c + jnp.log2(s)
    out_ref[...] = jnp.sum(logz - picked, axis=(0, 1), keepdims=True) * (
        _LN2 / _B)


def kernel(inputs, targets, features):
    x = inputs * (_INV_TEMP * _LOG2E)
    c = (jnp.sqrt(jnp.sum(x * x, axis=1, keepdims=True)) - _SHIFT)
    ft = features.T
    t = targets.reshape(_B, 1)
    blk = t // _BN
    grp = (t % _BN) // 128
    lane = t % 128
    tt = t - _NB * _BN
    s, p = pl.pallas_call(
        _stream_kernel,
        grid=(_NB + 1,),
        in_specs=[
            pl.BlockSpec((_B, _NF), lambda j: (0, 0)),
            pl.BlockSpec((_NF, _BN), lambda j: (0, jnp.minimum(j, _NB - 1))),
            pl.BlockSpec((_B, 1), lambda j: (0, 0)),
            pl.BlockSpec((_B, 1), lambda j: (0, 0)),
            pl.BlockSpec((_B, 1), lambda j: (0, 0)),
        ],
        out_specs=[
            pl.BlockSpec((_B, 128), lambda j: (0, 0)),
            pl.BlockSpec((_B, 128), lambda j: (0, 0)),
        ],
        out_shape=[
            jax.ShapeDtypeStruct((_B, 128), jnp.float32),
            jax.ShapeDtypeStruct((_B, 128), jnp.float32),
        ],
        scratch_shapes=[
            pltpu.VMEM((_B, _BN), jnp.float32),
        ],
    )(x, ft, c, blk, grp)
    out = pl.pallas_call(
        _finalize_kernel,
        grid=(1,),
        in_specs=[
            pl.BlockSpec((_B, _NF), lambda j: (0, 0)),
            pl.BlockSpec((_NF, _BN), lambda j: (0, _NB)),
            pl.BlockSpec((_B, 1), lambda j: (0, 0)),
            pl.BlockSpec((_B, 1), lambda j: (0, 0)),
            pl.BlockSpec((_B, 1), lambda j: (0, 0)),
            pl.BlockSpec((_B, 1), lambda j: (0, 0)),
            pl.BlockSpec((_B, 128), lambda j: (0, 0)),
            pl.BlockSpec((_B, 128), lambda j: (0, 0)),
        ],
        out_specs=pl.BlockSpec((1, 1), lambda j: (0, 0)),
        out_shape=jax.ShapeDtypeStruct((1, 1), jnp.float32),
    )(x, ft, c, tt, blk, lane, s, p)
    return out[0, 0]


# speedup vs baseline: 1.5598x; 1.5598x over previous
import functools

import jax
import jax.numpy as jnp
from jax import lax
from jax.experimental import pallas as pl
from jax.experimental.pallas import tpu as pltpu
from jax.experimental.pallas import tpu_sc as plsc

_NF = 64
_ND = 100000
_B = 1024
_BN = 2048
_NB = _ND // _BN
_TAIL = _ND - _NB * _BN
_NG = _BN // 128
_INV_TEMP = 20.0
_LN2 = 0.6931471805599453
_LOG2E = 1.4426950408889634
_SHIFT = 100.0

_NC = 2
_NS = 16
_L = 16
_NW = _NC * _NS
_BPW = _B // _NW


def _stream_kernel(x_ref, ft_ref, c_ref, key_ref, s_ref, p_ref, buf_ref):
    j = pl.program_id(0)

    @pl.when(j == 0)
    def _init():
        s_ref[...] = jnp.zeros((_B, 128), jnp.float32)
        p_ref[...] = jnp.zeros((_B, 128), jnp.float32)

    x = x_ref[...]

    prev = buf_ref[...]
    e = jnp.exp2(prev - c_ref[...])
    parts = [e[:, k * 128:(k + 1) * 128] for k in range(_NG)]
    while len(parts) > 1:
        parts = [a + b for a, b in zip(parts[::2], parts[1::2])]
    s_ref[...] += jnp.where(j > 0, parts[0], 0.0)

    key = key_ref[...]
    pacc = p_ref[...]
    for k in range(_NG):
        pacc = jnp.where(key == (j - 1) * _NG + k,
                         prev[:, k * 128:(k + 1) * 128], pacc)
    p_ref[...] = pacc

    ft = ft_ref[...]
    logits = lax.dot_general(
        x.astype(jnp.bfloat16), ft.astype(jnp.bfloat16),
        (((1,), (0,)), ((), ())),
        preferred_element_type=jnp.float32)
    buf_ref[...] = logits


def _finalize_kernel(x_ref, ft_ref, c_ref, tt_ref, blk_ref, lane_ref,
                     s_ref, p_ref, out_ref):
    x = x_ref[...]
    c = c_ref[...]

    ft = ft_ref[...]
    logits = lax.dot_general(
        x.astype(jnp.bfloat16), ft.astype(jnp.bfloat16),
        (((1,), (0,)), ((), ())),
        preferred_element_type=jnp.float32)
    col = lax.broadcasted_iota(jnp.int32, (_B, _BN), 1)
    e = jnp.where(col < _TAIL, jnp.exp2(logits - c), 0.0)
    lanes = s_ref[...] + sum(
        e[:, k * 128:(k + 1) * 128] for k in range(_NG))
    s = jnp.sum(lanes, axis=1, keepdims=True)

    tt = tt_ref[...]
    picked_tail = jnp.sum(jnp.where(col == tt, logits, 0.0),
                          axis=1, keepdims=True)
    lane128 = lax.broadcasted_iota(jnp.int32, (_B, 128), 1)
    picked_p = jnp.sum(jnp.where(lane128 == lane_ref[...], p_ref[...], 0.0),
                       axis=1, keepdims=True)
    picked = jnp.where(blk_ref[...] == _NB, picked_tail, picked_p)

    logz = c + jnp.log2(s)
    out_ref[...] = jnp.sum(logz - picked, axis=(0, 1), keepdims=True) * (
        _LN2 / _B)


def kernel(inputs, targets, features):
    x = inputs * (_INV_TEMP * _LOG2E)
    c = (jnp.sqrt(jnp.sum(x * x, axis=1, keepdims=True)) - _SHIFT)
    ft = features.T
    t = targets.reshape(_B, 1)
    blk = t // _BN
    grp = (t % _BN) // 128
    key = jnp.broadcast_to(blk * _NG + grp, (_B, 128)).astype(jnp.int32)
    lane = t % 128
    tt = t - _NB * _BN
    s, p = pl.pallas_call(
        _stream_kernel,
        grid=(_NB + 1,),
        in_specs=[
            pl.BlockSpec((_B, _NF), lambda j: (0, 0)),
            pl.BlockSpec((_NF, _BN), lambda j: (0, jnp.minimum(j, _NB - 1))),
            pl.BlockSpec((_B, 1), lambda j: (0, 0)),
            pl.BlockSpec((_B, 128), lambda j: (0, 0)),
        ],
        out_specs=[
            pl.BlockSpec((_B, 128), lambda j: (0, 0)),
            pl.BlockSpec((_B, 128), lambda j: (0, 0)),
        ],
        out_shape=[
            jax.ShapeDtypeStruct((_B, 128), jnp.float32),
            jax.ShapeDtypeStruct((_B, 128), jnp.float32),
        ],
        scratch_shapes=[
            pltpu.VMEM((_B, _BN), jnp.float32),
        ],
    )(x, ft, c, key)
    out = pl.pallas_call(
        _finalize_kernel,
        grid=(1,),
        in_specs=[
            pl.BlockSpec((_B, _NF), lambda j: (0, 0)),
            pl.BlockSpec((_NF, _BN), lambda j: (0, _NB)),
            pl.BlockSpec((_B, 1), lambda j: (0, 0)),
            pl.BlockSpec((_B, 1), lambda j: (0, 0)),
            pl.BlockSpec((_B, 1), lambda j: (0, 0)),
            pl.BlockSpec((_B, 1), lambda j: (0, 0)),
            pl.BlockSpec((_B, 128), lambda j: (0, 0)),
            pl.BlockSpec((_B, 128), lambda j: (0, 0)),
        ],
        out_specs=pl.BlockSpec((1, 1), lambda j: (0, 0)),
        out_shape=jax.ShapeDtypeStruct((1, 1), jnp.float32),
    )(x, ft, c, tt, blk, lane, s, p)
    return out[0, 0]


# speedup vs baseline: 1.5642x; 1.0028x over previous
import jax
import jax.numpy as jnp
from jax import lax
from jax.experimental import pallas as pl
from jax.experimental.pallas import tpu as pltpu

_NF = 64
_ND = 100000
_B = 1024
_BN = 2048
_NB = _ND // _BN
_TAIL = _ND - _NB * _BN
_NG = _BN // 128
_INV_TEMP = 20.0
_LN2 = 0.6931471805599453
_LOG2E = 1.4426950408889634
_SHIFT = 100.0

def _stream_kernel(x_ref, ft_ref, c_ref, key_ref, s_ref, p_ref, buf_ref):
    j = pl.program_id(0)

    @pl.when(j == 0)
    def _init():
        s_ref[...] = jnp.zeros((_B, 128), jnp.float32)
        p_ref[...] = jnp.zeros((_B, 128), jnp.float32)

    x = x_ref[...]

    prev = buf_ref[...]
    e = jnp.exp2(prev - c_ref[...])
    parts = [e[:, k * 128:(k + 1) * 128] for k in range(_NG)]
    while len(parts) > 1:
        parts = [a + b for a, b in zip(parts[::2], parts[1::2])]
    s_ref[...] += jnp.where(j > 0, parts[0], 0.0)

    key = key_ref[...]
    pacc = p_ref[...]
    for k in range(_NG):
        pacc = jnp.where(key == (j - 1) * _NG + k,
                         prev[:, k * 128:(k + 1) * 128], pacc)
    p_ref[...] = pacc

    ft = ft_ref[...]
    logits = lax.dot_general(
        x.astype(jnp.bfloat16), ft.astype(jnp.bfloat16),
        (((1,), (0,)), ((), ())),
        preferred_element_type=jnp.float32)
    buf_ref[...] = logits


def _finalize_kernel(x_ref, ft_ref, c_ref, tt_ref, blk_ref, lane_ref,
                     s_ref, p_ref, out_ref):
    x = x_ref[...]
    c = c_ref[...]

    ft = ft_ref[...]
    logits = lax.dot_general(
        x.astype(jnp.bfloat16), ft.astype(jnp.bfloat16),
        (((1,), (0,)), ((), ())),
        preferred_element_type=jnp.float32)
    col = lax.broadcasted_iota(jnp.int32, (_B, _BN), 1)
    e = jnp.where(col < _TAIL, jnp.exp2(logits - c), 0.0)
    lanes = s_ref[...] + sum(
        e[:, k * 128:(k + 1) * 128] for k in range(_NG))
    s = jnp.sum(lanes, axis=1, keepdims=True)

    tt = tt_ref[...]
    picked_tail = jnp.sum(jnp.where(col == tt, logits, 0.0),
                          axis=1, keepdims=True)
    lane128 = lax.broadcasted_iota(jnp.int32, (_B, 128), 1)
    picked_p = jnp.sum(jnp.where(lane128 == lane_ref[...], p_ref[...], 0.0),
                       axis=1, keepdims=True)
    picked = jnp.where(blk_ref[...] == _NB, picked_tail, picked_p)

    logz = c + jnp.log2(s)
    out_ref[...] = jnp.sum(logz - picked, axis=(0, 1), keepdims=True) * (
        _LN2 / _B)


def kernel(inputs, targets, features):
    x = inputs * (_INV_TEMP * _LOG2E)
    c = (jnp.sqrt(jnp.sum(x * x, axis=1, keepdims=True)) - _SHIFT)
    ft = features.T
    t = targets.reshape(_B, 1)
    blk = t // _BN
    grp = (t % _BN) // 128
    key = jnp.broadcast_to(blk * _NG + grp, (_B, 128)).astype(jnp.int32)
    lane = t % 128
    tt = t - _NB * _BN
    s, p = pl.pallas_call(
        _stream_kernel,
        grid=(_NB + 1,),
        in_specs=[
            pl.BlockSpec((_B, _NF), lambda j: (0, 0)),
            pl.BlockSpec((_NF, _BN), lambda j: (0, jnp.minimum(j, _NB - 1))),
            pl.BlockSpec((_B, 1), lambda j: (0, 0)),
            pl.BlockSpec((_B, 128), lambda j: (0, 0)),
        ],
        out_specs=[
            pl.BlockSpec((_B, 128), lambda j: (0, 0)),
            pl.BlockSpec((_B, 128), lambda j: (0, 0)),
        ],
        out_shape=[
            jax.ShapeDtypeStruct((_B, 128), jnp.float32),
            jax.ShapeDtypeStruct((_B, 128), jnp.float32),
        ],
        scratch_shapes=[
            pltpu.VMEM((_B, _BN), jnp.float32),
        ],
    )(x, ft, c, key)
    out = pl.pallas_call(
        _finalize_kernel,
        grid=(1,),
        in_specs=[
            pl.BlockSpec((_B, _NF), lambda j: (0, 0)),
            pl.BlockSpec((_NF, _BN), lambda j: (0, _NB)),
            pl.BlockSpec((_B, 1), lambda j: (0, 0)),
            pl.BlockSpec((_B, 1), lambda j: (0, 0)),
            pl.BlockSpec((_B, 1), lambda j: (0, 0)),
            pl.BlockSpec((_B, 1), lambda j: (0, 0)),
            pl.BlockSpec((_B, 128), lambda j: (0, 0)),
            pl.BlockSpec((_B, 128), lambda j: (0, 0)),
        ],
        out_specs=pl.BlockSpec((1, 1), lambda j: (0, 0)),
        out_shape=jax.ShapeDtypeStruct((1, 1), jnp.float32),
    )(x, ft, c, tt, blk, lane, s, p)
    return out[0, 0]
